# Initial kernel scaffold; baseline (speedup 1.0000x reference)
import jax
import jax.numpy as jnp
from jax.experimental import pallas as pl


def kernel(support_xf, support_y, query_xf, query_y, unlabeled_xf):
    raise NotImplementedError("write your pallas kernel here")


# speedup vs baseline: 14.8413x; 14.8413x over previous
import functools

import jax
import jax.numpy as jnp
from jax import lax
from jax.experimental import pallas as pl
from jax.experimental.pallas import tpu as pltpu
from jax.experimental.pallas import tpu_sc as plsc

_NW = 5
_MS = 125
_MU = 5000
_MUP = 5120
_CAP = 640
_W2 = 128 + _CAP
_B = 4
_Q = 75
_QT = 5
_C = 64
_EPS = 1e-12


def _rownorm(x):
    n = jnp.sqrt(jnp.sum(x * x, axis=1, keepdims=True))
    return x / jnp.maximum(n, _EPS)


def _s1_body(u_ref, s_ref, out_ref):
    u = u_ref[0]
    s = s_ref[0]
    un = _rownorm(u)
    sn = _rownorm(s)
    S = lax.dot_general(un, sn, (((1,), (1,)), ((), ())),
                        preferred_element_type=jnp.float32)
    row1 = lax.broadcasted_iota(jnp.int32, (_MUP, 1), 0)
    col1 = lax.broadcasted_iota(jnp.int32, (1, _NW * _MS + 15), 1)
    neg = jnp.float32(-jnp.inf)
    S = jnp.where(col1 < _NW * _MS, S, neg)
    S = jnp.where(row1 < _MU, S, neg)
    u_near = jnp.argmax(S, axis=1, keepdims=True).astype(jnp.int32)
    s_near = jnp.argmax(S, axis=0, keepdims=True).astype(jnp.int32)
    best = jnp.max(S[:, 0:_MS], axis=1, keepdims=True)
    bi = jnp.zeros((_MUP, 1), jnp.int32)
    for n in range(1, _NW):
        cm = jnp.max(S[:, n * _MS:(n + 1) * _MS], axis=1, keepdims=True)
        upd = cm > best
        bi = jnp.where(upd, n, bi)
        best = jnp.where(upd, cm, best)
    oh = u_near == col1
    mutual = jnp.sum(jnp.where(oh, s_near, 0), axis=1, keepdims=True)
    selected = (mutual == row1) & (row1 < _MU)
    selcls = jnp.where(selected, bi, -1)
    out_ref[...] = selcls[None]


def _s1_call(u_pad, s_pad):
    return pl.pallas_call(
        _s1_body,
        grid=(_B,),
        in_specs=[
            pl.BlockSpec((1, _MUP, _C), lambda b: (b, 0, 0)),
            pl.BlockSpec((1, _NW * _MS + 15, _C), lambda b: (b, 0, 0)),
        ],
        out_specs=pl.BlockSpec((1, _MUP, 1), lambda b: (b, 0, 0)),
        out_shape=jax.ShapeDtypeStruct((_B, _MUP, 1), jnp.int32),
    )(u_pad, s_pad)


def _sc_body(selcls_hbm, uflat_hbm, dual_hbm, counts_hbm,
             selcls_v, idx2_v, rows_v, cnt_v, sem):
    wid = lax.axis_index("s") * 2 + lax.axis_index("c")

    @pl.when(wid < _B * _NW)
    def _():
        b = wid // _NW
        n = wid % _NW
        pltpu.sync_copy(selcls_hbm.at[b], selcls_v)
        zrow = b * _MUP + _MU

        def init_body(i, carry):
            for j in range(_CAP // 128):
                idx2_v[j, pl.ds(i * 16, 16)] = jnp.full((16,), zrow, jnp.int32)
            return carry

        lax.fori_loop(0, 8, init_body, jnp.int32(0))
        lane = lax.iota(jnp.int32, 16)
        nvec = jnp.full((16,), n, jnp.int32)
        one = jnp.full((16,), 1, jnp.int32)
        zero = jnp.full((16,), 0, jnp.int32)
        seven = jnp.full((16,), 7, jnp.int32)
        c127 = jnp.full((16,), 127, jnp.int32)

        def scan_body(i, cnt):
            chunk = selcls_v[pl.ds(i * 16, 16)]
            m = chunk == nvec
            mi = jnp.where(m, one, zero)
            cntv = jnp.full((16,), cnt, jnp.int32)
            pos = jnp.maximum(cntv + plsc.cumsum(mi) - one, zero)
            gidx = lane + jnp.full((16,), i * 16 + b * _MUP, jnp.int32)
            plsc.store_scatter(idx2_v, [pos >> seven, pos & c127], gidx,
                               mask=m)
            return cnt + jnp.sum(mi)

        cnt = lax.fori_loop(0, _MUP // 16, scan_body, jnp.int32(0))
        for j in range(_CAP // 128):
            pltpu.async_copy(uflat_hbm.at[idx2_v.at[j]],
                             rows_v.at[pl.ds(j * 128, 128)], sem).wait()
        cnt_v[...] = jnp.full((16,), cnt, jnp.int32)
        pltpu.sync_copy(cnt_v, counts_hbm.at[wid])
        pltpu.sync_copy(rows_v, dual_hbm.at[wid])


@functools.lru_cache(maxsize=1)
def _sc_compact_fn():
    mesh = plsc.VectorSubcoreMesh(core_axis_name="c", subcore_axis_name="s")
    return pl.kernel(
        _sc_body,
        mesh=mesh,
        out_type=[
            jax.ShapeDtypeStruct((_B * _NW, _CAP, 128), jnp.float32),
            jax.ShapeDtypeStruct((32, 16), jnp.int32),
        ],
        scratch_types=[
            pltpu.VMEM((_MUP,), jnp.int32),
            pltpu.VMEM((5, 128), jnp.int32),
            pltpu.VMEM((_CAP, 128), jnp.float32),
            pltpu.VMEM((16,), jnp.int32),
            pltpu.SemaphoreType.DMA,
        ],
        compiler_params=pltpu.CompilerParams(needs_layout_passes=False),
    )


def _s2_body(counts_ref, qy_ref, qf_ref, sup_ref, dual_ref, out_ref):
    bi = pl.program_id(0)
    qt = pl.program_id(1)
    L = jnp.max(counts_ref[...])
    q = qf_ref[0].reshape(_QT * 32, _C)
    qn = _rownorm(q)
    colj = lax.broadcasted_iota(jnp.int32, (1, _W2), 1)
    rowp = lax.broadcasted_iota(jnp.int32, (_QT * 32, 1), 0)
    padrow = (rowp % 32) >= 25
    neg = jnp.float32(-jnp.inf)
    valid = (colj < _MS) | ((colj >= 128) & (colj < 128 + L))
    Sn_list = []
    rm_list = []
    for n in range(_NW):
        scn = jnp.concatenate([sup_ref[0, n], dual_ref[0, n, :, 0:_C]],
                              axis=0)
        scn = _rownorm(scn)
        Sn = lax.dot_general(qn, scn, (((1,), (1,)), ((), ())),
                             preferred_element_type=jnp.float32)
        Sn = jnp.where(valid, Sn, neg)
        Sn = jnp.where(padrow, neg, Sn)
        rm_list.append(jnp.max(Sn, axis=1, keepdims=True))
        Sn_list.append(Sn)
    S = jnp.concatenate(Sn_list, axis=1)
    colg = lax.broadcasted_iota(jnp.int32, (1, _NW * _W2), 1)
    row32 = lax.broadcasted_iota(jnp.int32, (32, 1), 0)
    nll_sum = jnp.float32(0.0)
    for k in range(_QT):
        Sk = S[k * 32:(k + 1) * 32, :]
        cmax = jnp.max(Sk, axis=0, keepdims=True)
        carg = jnp.min(jnp.where(Sk == cmax, row32, 99), axis=0,
                       keepdims=True)
        qnear = jnp.argmax(Sk, axis=1, keepdims=True).astype(jnp.int32)
        mutual = jnp.sum(jnp.where(qnear == colg, carg, 0), axis=1,
                         keepdims=True)
        qmask = (mutual == row32) & (row32 < 25)
        logits = []
        for n in range(_NW):
            rmk = rm_list[n][k * 32:(k + 1) * 32, :]
            qv = jnp.sum(jnp.where(qmask, rmk, 0.0))
            logits.append(qv / 2.0)
        m = logits[0]
        for n in range(1, _NW):
            m = jnp.maximum(m, logits[n])
        sexp = jnp.float32(0.0)
        for n in range(_NW):
            sexp = sexp + jnp.exp(logits[n] - m)
        lse = m + jnp.log(sexp)
        y = qy_ref[bi, qt * _QT + k]
        pick = jnp.float32(0.0)
        for n in range(_NW):
            pick = pick + jnp.where(y == n, logits[n], 0.0)
        nll_sum = nll_sum + (lse - pick)

    @pl.when((bi == 0) & (qt == 0))
    def _():
        out_ref[...] = jnp.zeros((1, 1), jnp.float32)

    out_ref[...] = out_ref[...] + (nll_sum / (_B * _Q)).reshape(1, 1)


def _s2_call(counts, qy, qf_pad, sup_pad, dual):
    return pl.pallas_call(
        _s2_body,
        grid=(_B, _Q // _QT),
        in_specs=[
            pl.BlockSpec((_B, _NW), lambda b, t: (0, 0)),
            pl.BlockSpec(memory_space=pltpu.SMEM),
            pl.BlockSpec((1, _QT, 32, _C), lambda b, t: (b, t, 0, 0)),
            pl.BlockSpec((1, _NW, 128, _C), lambda b, t: (b, 0, 0, 0)),
            pl.BlockSpec((1, _NW, _CAP, 128), lambda b, t: (b, 0, 0, 0)),
        ],
        out_specs=pl.BlockSpec((1, 1), lambda b, t: (0, 0)),
        out_shape=jax.ShapeDtypeStruct((1, 1), jnp.float32),
    )(counts, qy, qf_pad, sup_pad, dual)


def kernel(support_xf, support_y, query_xf, query_y, unlabeled_xf):
    u_feats = unlabeled_xf.reshape(_B, 200, _C, 25).transpose(0, 1, 3, 2)
    u_feats = u_feats.reshape(_B, _MU, _C)
    u_pad = jnp.pad(u_feats, ((0, 0), (0, _MUP - _MU), (0, 0)))
    sup4 = support_xf.reshape(_B, _NW, _NW, _C, 25).transpose(0, 1, 3, 2, 4)
    sup4 = sup4.reshape(_B, _NW, _C, _MS)
    s_cols = sup4.transpose(0, 1, 3, 2).reshape(_B, _NW * _MS, _C)
    s_pad = jnp.pad(s_cols, ((0, 0), (0, 15), (0, 0)))

    selcls = _s1_call(u_pad, s_pad).reshape(_B, _MUP)
    uflat = jnp.pad(u_pad, ((0, 0), (0, 0), (0, 128 - _C)))
    uflat = uflat.reshape(_B * _MUP, 128)
    dual20, counts32 = _sc_compact_fn()(selcls, uflat)
    dual = dual20.reshape(_B, _NW, _CAP, 128)
    counts = counts32[:_B * _NW, 0].reshape(_B, _NW)

    sup_rows = sup4.transpose(0, 1, 3, 2)
    sup_pad = jnp.pad(sup_rows, ((0, 0), (0, 0), (0, 3), (0, 0)))
    qf = query_xf.reshape(_B, _Q, _C, 25).transpose(0, 1, 3, 2)
    qf_pad = jnp.pad(qf, ((0, 0), (0, 0), (0, 7), (0, 0)))

    out = _s2_call(counts, query_y, qf_pad, sup_pad, dual)
    return out.reshape(())
